# baseline (speedup 1.0000x reference)
import functools

import jax
import jax.numpy as jnp
from jax import lax
from jax.experimental import pallas as pl
from jax.experimental.pallas import tpu as pltpu
from jax.experimental.pallas import tpu_sc as plsc

_IDX_MINOR = 640
_NBUF = 2


@functools.lru_cache(maxsize=None)
def _build(B, V, D, chunk, idx_minor):
  NC, NS = 2, 16
  NW = NC * NS
  n_stream = chunk // idx_minor
  b_per_w = B // NW
  n_chunk = b_per_w // chunk
  idx_rows_w = b_per_w // idx_minor
  assert n_chunk % _NBUF == 0

  mesh = plsc.VectorSubcoreMesh(core_axis_name="c", subcore_axis_name="s")

  @functools.partial(
      pl.kernel,
      mesh=mesh,
      compiler_params=pltpu.CompilerParams(use_tc_tiling_on_sc=False),
      out_type=jax.ShapeDtypeStruct((B, D), jnp.float32),
      scratch_types=[
          pltpu.VMEM((idx_rows_w, idx_minor), jnp.int32),
          pltpu.VMEM((_NBUF * chunk, D), jnp.float32),
          pltpu.SemaphoreType.DMA((_NBUF,)),
          pltpu.SemaphoreType.DMA((_NBUF,)),
      ],
  )
  def gather_kernel(idx_hbm, table_hbm, out_hbm, idx_v, rows_v, gsem, osem):
    wid = lax.axis_index("s") * NC + lax.axis_index("c")
    out_base_w = wid * b_per_w

    pltpu.sync_copy(idx_hbm.at[pl.ds(wid * idx_rows_w, idx_rows_w)], idx_v)

    def fire_gathers(g, b):
      for j in range(n_stream):
        pltpu.async_copy(
            table_hbm.at[idx_v.at[g * n_stream + j]],
            rows_v.at[pl.ds(b * chunk + j * idx_minor, idx_minor)],
            gsem.at[b])

    def drain_gathers(b):
      pltpu.make_async_copy(table_hbm.at[pl.ds(0, chunk)],
                            rows_v.at[pl.ds(b * chunk, chunk)],
                            gsem.at[b]).wait()

    def wait_outwrite(b):
      pltpu.make_async_copy(rows_v.at[pl.ds(b * chunk, chunk)],
                            out_hbm.at[pl.ds(out_base_w, chunk)],
                            osem.at[b]).wait()

    for b in range(_NBUF):
      fire_gathers(b, b)

    def body(t, carry):
      for b in range(_NBUF):
        g = _NBUF * t + b
        drain_gathers(b)
        pltpu.async_copy(rows_v.at[pl.ds(b * chunk, chunk)],
                         out_hbm.at[pl.ds(out_base_w + g * chunk, chunk)],
                         osem.at[b])
      for b in range(_NBUF):
        g_next = _NBUF * t + b + _NBUF

        @pl.when(g_next < n_chunk)
        def _():
          wait_outwrite(b)
          fire_gathers(g_next, b)

      return carry

    lax.fori_loop(0, n_chunk // _NBUF, body, 0)
    for b in range(_NBUF):
      wait_outwrite(b)

  return gather_kernel


def kernel(input_sentence, word_embedding_weight):
  S, W = input_sentence.shape
  V, D = word_embedding_weight.shape
  B = S * W
  idx2d = input_sentence.reshape(B // _IDX_MINOR, _IDX_MINOR).astype(jnp.int32)
  fn = _build(B, V, D, 640, _IDX_MINOR)
  out = fn(idx2d, word_embedding_weight)
  return out.reshape(S, W, D)


# speedup vs baseline: 1.1382x; 1.1382x over previous
import functools

import jax
import jax.numpy as jnp
from jax import lax
from jax.experimental import pallas as pl
from jax.experimental.pallas import tpu as pltpu
from jax.experimental.pallas import tpu_sc as plsc

_IDX_MINOR = 640
_NBUF = 2


def _detile_body(x_ref, o_ref):
  x = x_ref[...]
  y = x.T
  half = y.shape[0] // 2
  o_ref[:, 0:64] = y[0:half]
  o_ref[:, 64:128] = y[half:]


@functools.lru_cache(maxsize=None)
def _build_detile(V, D, cols):
  grid = -(-V // cols)
  return pl.pallas_call(
      _detile_body,
      grid=(grid,),
      in_specs=[pl.BlockSpec((D, cols), lambda i: (0, i))],
      out_specs=pl.BlockSpec((cols // 2, 128), lambda i: (i, 0)),
      out_shape=jax.ShapeDtypeStruct((grid * cols // 2, 128), jnp.float32),
  )


@functools.lru_cache(maxsize=None)
def _build_gather(B, V, D, chunk, idx_minor):
  NC, NS = 2, 16
  NW = NC * NS
  n_stream = chunk // idx_minor
  b_per_w = B // NW
  n_chunk = b_per_w // chunk
  idx_rows_w = b_per_w // idx_minor
  assert n_chunk % _NBUF == 0

  mesh = plsc.VectorSubcoreMesh(core_axis_name="c", subcore_axis_name="s")

  @functools.partial(
      pl.kernel,
      mesh=mesh,
      compiler_params=pltpu.CompilerParams(use_tc_tiling_on_sc=False),
      out_type=jax.ShapeDtypeStruct((B, D), jnp.float32),
      scratch_types=[
          pltpu.VMEM((idx_rows_w, idx_minor), jnp.int32),
          pltpu.VMEM((_NBUF * chunk, D), jnp.float32),
          pltpu.SemaphoreType.DMA((_NBUF,)),
          pltpu.SemaphoreType.DMA((_NBUF,)),
      ],
  )
  def gather_kernel(idx_hbm, table_hbm, out_hbm, idx_v, rows_v, gsem, osem):
    wid = lax.axis_index("s") * NC + lax.axis_index("c")
    out_base_w = wid * b_per_w

    pltpu.sync_copy(idx_hbm.at[pl.ds(wid * idx_rows_w, idx_rows_w)], idx_v)

    def fire_gathers(g, b):
      for j in range(n_stream):
        pltpu.async_copy(
            table_hbm.at[idx_v.at[g * n_stream + j]],
            rows_v.at[pl.ds(b * chunk + j * idx_minor, idx_minor)],
            gsem.at[b])

    def drain_gathers(b):
      pltpu.make_async_copy(table_hbm.at[pl.ds(0, chunk)],
                            rows_v.at[pl.ds(b * chunk, chunk)],
                            gsem.at[b]).wait()

    def wait_outwrite(b):
      pltpu.make_async_copy(rows_v.at[pl.ds(b * chunk, chunk)],
                            out_hbm.at[pl.ds(out_base_w, chunk)],
                            osem.at[b]).wait()

    for b in range(_NBUF):
      fire_gathers(b, b)

    def body(t, carry):
      for b in range(_NBUF):
        g = _NBUF * t + b
        drain_gathers(b)
        pltpu.async_copy(rows_v.at[pl.ds(b * chunk, chunk)],
                         out_hbm.at[pl.ds(out_base_w + g * chunk, chunk)],
                         osem.at[b])
      for b in range(_NBUF):
        g_next = _NBUF * t + b + _NBUF

        @pl.when(g_next < n_chunk)
        def _():
          wait_outwrite(b)
          fire_gathers(g_next, b)

      return carry

    lax.fori_loop(0, n_chunk // _NBUF, body, 0)
    for b in range(_NBUF):
      wait_outwrite(b)

  return gather_kernel


def _retile_body(x_ref, o_ref):
  x = x_ref[...][0]
  half = x.shape[0]
  o_ref[0, :, 0:half] = x[:, 0:64].T
  o_ref[0, :, half:2 * half] = x[:, 64:128].T


@functools.lru_cache(maxsize=None)
def _build_retile(S, W, D, sblk):
  grid_s = S // sblk
  return pl.pallas_call(
      _retile_body,
      grid=(W, grid_s),
      in_specs=[
          pl.BlockSpec((1, sblk * D // 128, 128), lambda w, j: (w, j, 0))
      ],
      out_specs=pl.BlockSpec((1, D, sblk), lambda w, j: (w, 0, j)),
      out_shape=jax.ShapeDtypeStruct((W, D, S), jnp.float32),
  )


_COLS = 2048
_SBLK = 4096


def kernel(input_sentence, word_embedding_weight):
  S, W = input_sentence.shape
  V, D = word_embedding_weight.shape
  B = S * W

  wt = word_embedding_weight.T
  t2d = _build_detile(V, D, _COLS)(wt)
  Vp = t2d.shape[0] * 2
  table_rm = t2d.reshape(Vp * D // 128 * 128).reshape(Vp, D)

  idxT = input_sentence.T.astype(jnp.int32)
  blk = idxT.reshape(W, S // _SBLK, 2, _SBLK // 2)
  idx_m = blk.transpose(0, 1, 3, 2).reshape(B)
  u = (jnp.bitwise_and(idx_m, -_COLS)
       + ((idx_m & (_COLS // 2 - 1)) << 1)
       + ((idx_m >> 10) & 1))
  idx2d = u.reshape(B // _IDX_MINOR, _IDX_MINOR)

  out_sc = _build_gather(B, Vp, D, 640, _IDX_MINOR)(idx2d, table_rm)

  out_v = out_sc.reshape(B * D).reshape(W, S * D // 128, 128)
  out_t = _build_retile(S, W, D, _SBLK)(out_v)
  return out_t.transpose(2, 0, 1)
